# Initial kernel scaffold; baseline (speedup 1.0000x reference)
import jax
import jax.numpy as jnp
from jax.experimental import pallas as pl


def kernel(visit_emb, visit_offset, ccs_emb, ccs_offset, icd_emb, icd_offset, cn_w1, cn_b1, cn_w2, cn_b2, t_w1, t_b1, t_w2, t_b2, graph):
    raise NotImplementedError("write your pallas kernel here")


# speedup vs baseline: 7.6026x; 7.6026x over previous
import functools

import jax
import jax.numpy as jnp
from jax import lax
from jax.experimental import pallas as pl
from jax.experimental.pallas import tpu as pltpu
from jax.experimental.pallas import tpu_sc as plsc

NV = 6000
NC = 300
NI = 3700
NN = NV + NC + NI
E = 320000
DIM = 128
LANES = 16

NCORE = 2
NSUB = 16
NWORK = NCORE * NSUB
RPW = 315
NMAXW = (NV + NC) // RPW
NP = NWORK * RPW

CE = 2560
NVREG = CE // LANES
NCHUNK = E // CE
FLUSH = 2048
CAP = E + 2 * FLUSH
B = 32
TRASH = RPW
BUFSZ = FLUSH + 3 * LANES
TRASHSLOT = FLUSH + 2 * LANES


def _wid():
    return lax.axis_index("s") * NCORE + lax.axis_index("c")


def _partition_body(head_hbm, tail_hbm, tails_out, lheads_out, counts_out,
                    hbuf, tbuf, bt, bl, cstage):
    wid = _wid()
    lo = wid * RPW
    hbm_base = wid * CAP

    def flush_if_full(cur, hcur):
        def do_flush(args):
            cur, hcur = args
            pltpu.sync_copy(bt.at[pl.ds(0, FLUSH)],
                            tails_out.at[pl.ds(pl.multiple_of(hbm_base + hcur, FLUSH), FLUSH)])
            pltpu.sync_copy(bl.at[pl.ds(0, FLUSH)],
                            lheads_out.at[pl.ds(pl.multiple_of(hbm_base + hcur, FLUSH), FLUSH)])
            bt[pl.ds(0, 16)] = bt[pl.ds(FLUSH, 16)]
            bt[pl.ds(16, 16)] = bt[pl.ds(FLUSH + 16, 16)]
            bl[pl.ds(0, 16)] = bl[pl.ds(FLUSH, 16)]
            bl[pl.ds(16, 16)] = bl[pl.ds(FLUSH + 16, 16)]
            return (cur - FLUSH, hcur + FLUSH)

        return lax.cond(cur >= FLUSH, do_flush, lambda a: a, (cur, hcur))

    def chunk_body(i, carry):
        pltpu.sync_copy(head_hbm.at[pl.ds(i * CE, CE)], hbuf)
        pltpu.sync_copy(tail_hbm.at[pl.ds(i * CE, CE)], tbuf)

        def vreg_body(j, c2):
            cur, hcur = c2
            h = hbuf[pl.ds(j * LANES, LANES)]
            t = tbuf[pl.ds(j * LANES, LANES)]
            m = (h >= lo) & (h < lo + RPW)
            cs = plsc.cumsum(m.astype(jnp.int32))
            pos = jnp.where(m, cur + cs - 1, TRASHSLOT)
            plsc.store_scatter(bt, [pos], t)
            plsc.store_scatter(bl, [pos], h - lo)
            return flush_if_full(cur + jnp.max(cs), hcur)

        return lax.fori_loop(0, NVREG, vreg_body, carry)

    cur, hcur = lax.fori_loop(0, NCHUNK, chunk_body,
                              (jnp.int32(0), jnp.int32(0)))

    trash_t = jnp.zeros((LANES,), jnp.int32)
    trash_l = jnp.full((LANES,), TRASH, jnp.int32)
    for k in range(2):
        bt[pl.ds(cur + k * LANES, LANES)] = trash_t
        bl[pl.ds(cur + k * LANES, LANES)] = trash_l
    cur, hcur = flush_if_full(cur + 2 * LANES, hcur)
    pltpu.sync_copy(bt.at[pl.ds(0, FLUSH)],
                    tails_out.at[pl.ds(pl.multiple_of(hbm_base + hcur, FLUSH), FLUSH)])
    pltpu.sync_copy(bl.at[pl.ds(0, FLUSH)],
                    lheads_out.at[pl.ds(pl.multiple_of(hbm_base + hcur, FLUSH), FLUSH)])
    total = hcur + cur - 2 * LANES
    padded = ((total + B - 1) // B) * B
    cstage[pl.ds(0, LANES)] = jnp.full((LANES,), 0, jnp.int32) + padded
    pltpu.sync_copy(cstage, counts_out.at[pl.ds(pl.multiple_of(wid * LANES, LANES), LANES)])


def _hop_body(tails_l, lheads_l, counts, table, out_flat,
              idx_v, lh_v, rows_v, cvm, accf, sem):
    wid = _wid()
    lo = wid * RPW
    is_max = wid < NMAXW
    sgn = jnp.where(is_max, jnp.float32(1.0), jnp.float32(-1.0))
    s_splat = jnp.zeros((LANES,), jnp.float32) + sgn
    pos_splat = s_splat > 0.0

    neg = jnp.full((LANES,), -jnp.inf, jnp.float32)

    def initb(r, _):
        for c in range(DIM // LANES):
            accf[pl.ds(r * DIM + c * LANES, LANES)] = neg
        return 0

    lax.fori_loop(0, RPW + 1, initb, 0)

    pltpu.sync_copy(counts.at[pl.ds(pl.multiple_of(wid * LANES, LANES), LANES)], cvm)
    cnt = jnp.max(cvm[pl.ds(0, LANES)])
    nb = cnt // B

    def batch(b, _):
        base = pl.multiple_of(wid * CAP + b * B, B)
        pltpu.sync_copy(tails_l.at[pl.ds(base, B)], idx_v)
        pltpu.sync_copy(lheads_l.at[pl.ds(base, B)], lh_v)
        pltpu.async_copy(table.at[idx_v], rows_v, sem).wait()
        lanes = lax.iota(jnp.int32, LANES)

        def edge(k, _):
            lv = lh_v[pl.ds((k // LANES) * LANES, LANES)]
            h = jnp.max(jnp.where(lanes == k % LANES, lv, -1))
            rbase = h * DIM
            for c in range(DIM // LANES):
                sl = pl.ds(rbase + c * LANES, LANES)
                r = rows_v[k, pl.ds(c * LANES, LANES)] * s_splat
                accf[sl] = jnp.maximum(accf[sl], r)
            return 0

        lax.fori_loop(0, B, edge, 0)
        return 0

    lax.fori_loop(0, nb, batch, 0)

    big_neg = jnp.full((LANES,), -3.0e38, jnp.float32)

    def fin(r, _):
        for c in range(DIM // LANES):
            sl = pl.ds(r * DIM + c * LANES, LANES)
            v = accf[sl]
            posv = jnp.maximum(v, 0.0)
            minv = jnp.where(v < big_neg, 0.0,
                             jnp.maximum(-v, 0.0))
            accf[sl] = jnp.where(pos_splat, posv, minv)
        return 0

    lax.fori_loop(0, RPW, fin, 0)
    pltpu.sync_copy(accf.at[pl.ds(0, RPW * DIM)],
                    out_flat.at[pl.ds(pl.multiple_of(lo * DIM, 64), RPW * DIM)])


_MESH = plsc.VectorSubcoreMesh(core_axis_name="c", subcore_axis_name="s")

_partition = functools.partial(
    pl.kernel,
    mesh=_MESH,
    compiler_params=pltpu.CompilerParams(needs_layout_passes=False),
    out_type=(
        jax.ShapeDtypeStruct((NWORK * CAP,), jnp.int32),
        jax.ShapeDtypeStruct((NWORK * CAP,), jnp.int32),
        jax.ShapeDtypeStruct((NWORK * LANES,), jnp.int32),
    ),
    scratch_types=[
        pltpu.VMEM((CE,), jnp.int32),
        pltpu.VMEM((CE,), jnp.int32),
        pltpu.VMEM((BUFSZ,), jnp.int32),
        pltpu.VMEM((BUFSZ,), jnp.int32),
        pltpu.VMEM((LANES,), jnp.int32),
    ],
)(_partition_body)

_hop = functools.partial(
    pl.kernel,
    mesh=_MESH,
    compiler_params=pltpu.CompilerParams(needs_layout_passes=False),
    out_type=jax.ShapeDtypeStruct((NP * DIM,), jnp.float32),
    scratch_types=[
        pltpu.VMEM((B,), jnp.int32),
        pltpu.VMEM((B,), jnp.int32),
        pltpu.VMEM((B, DIM), jnp.float32),
        pltpu.VMEM((LANES,), jnp.int32),
        pltpu.VMEM(((RPW + 1) * DIM,), jnp.float32),
        pltpu.SemaphoreType.DMA,
    ],
)(_hop_body)


def kernel(visit_emb, visit_offset, ccs_emb, ccs_offset, icd_emb, icd_offset,
           cn_w1, cn_b1, cn_w2, cn_b2, t_w1, t_b1, t_w2, t_b2, graph):
    head = graph[0]
    tail = graph[1]

    tails_l, lheads_l, counts = _partition(head, tail)

    off0 = jnp.concatenate([visit_offset, ccs_offset, icd_offset], axis=0)
    off0 = jnp.concatenate(
        [off0, jnp.zeros((NP - NN, DIM), jnp.float32)], axis=0)

    off1 = _hop(tails_l, lheads_l, counts, off0)
    off2 = _hop(tails_l, lheads_l, counts, off1.reshape(NP, DIM))

    out_off = off2.reshape(NP, DIM)[:NV]
    out_emb = jnp.zeros((NV, DIM), jnp.float32)
    return out_emb, out_off


# speedup vs baseline: 13.4572x; 1.7701x over previous
import functools

import jax
import jax.numpy as jnp
from jax import lax
from jax.experimental import pallas as pl
from jax.experimental.pallas import tpu as pltpu
from jax.experimental.pallas import tpu_sc as plsc

NV = 6000
NC = 300
NI = 3700
NN = NV + NC + NI
E = 320000
DIM = 128
LANES = 16
NCH = DIM // LANES

NCORE = 2
NSUB = 16
NWORK = NCORE * NSUB
RPW = 315
NMAXW = (NV + NC) // RPW
NP = NWORK * RPW

CE = 2000
NVREG = CE // LANES
NCHUNK = E // CE
FLUSH = 2048
RING = 8192
RINGM = RING - 1
TR = RING
CAP = 158 * FLUSH
B = 128
PB = 2 * B
TRASH = RPW
SHIFT = 14


def _wid():
    return lax.axis_index("s") * NCORE + lax.axis_index("c")


def _partition_body(head_hbm, tail_hbm, pk_out, counts_out,
                    buf, cstage, h0, h1, t0, t1, sem0, sem1):
    hb, tb, sems = [h0, h1], [t0, t1], [sem0, sem1]
    wid = _wid()
    lo = wid * RPW
    hbm_base = wid * CAP
    lanes = lax.iota(jnp.int32, LANES)

    def issue(i, s):
        off = pl.multiple_of(jnp.minimum(i, NCHUNK - 1) * CE, CE)
        pltpu.async_copy(head_hbm.at[pl.ds(off, CE)], hb[s], sems[s])
        pltpu.async_copy(tail_hbm.at[pl.ds(off, CE)], tb[s], sems[s])

    def wait(s):
        pltpu.make_async_copy(head_hbm.at[pl.ds(0, CE)], hb[s], sems[s]).wait()
        pltpu.make_async_copy(tail_hbm.at[pl.ds(0, CE)], tb[s], sems[s]).wait()

    def flush_block(flushed):
        roff = pl.multiple_of(flushed & RINGM, FLUSH)
        pltpu.sync_copy(
            buf.at[pl.ds(roff, FLUSH)],
            pk_out.at[pl.ds(pl.multiple_of(hbm_base + flushed, FLUSH), FLUSH)])
        return flushed + FLUSH

    def chunk(i, s, carry):
        cur_v, flushed = carry
        wait(s)

        def vreg(j, cur_v):
            h = hb[s][pl.ds(j * LANES, LANES)]
            t = tb[s][pl.ds(j * LANES, LANES)]
            m = (h >= lo) & (h < lo + RPW)
            cs = plsc.cumsum(m.astype(jnp.int32))
            pos = jnp.where(m, (cur_v + cs - 1) & RINGM, TR)
            packed = t | ((h - lo) << SHIFT)
            plsc.store_scatter(buf, [pos], packed)
            return cur_v + plsc.all_reduce_population_count(m)

        cur_v = lax.fori_loop(0, NVREG, vreg, cur_v)
        issue(i + 2, s)
        cur_sc = jnp.max(cur_v)
        flushed = lax.cond(cur_sc - flushed >= FLUSH, flush_block,
                           lambda f: f, flushed)
        flushed = lax.cond(cur_sc - flushed >= FLUSH, flush_block,
                           lambda f: f, flushed)
        return (cur_v, flushed)

    issue(0, 0)
    issue(1, 1)

    def gbody(g, carry):
        for s in (0, 1):
            carry = chunk(g * 2 + s, s, carry)
        return carry

    cur_v, flushed = lax.fori_loop(
        0, NCHUNK // 2, gbody,
        (jnp.zeros((LANES,), jnp.int32), jnp.int32(0)))
    wait(0)
    wait(1)

    trash_pk = jnp.full((LANES,), TRASH << SHIFT, jnp.int32)
    for k in range(PB // LANES):
        pos = (cur_v + (k * LANES) + lanes) & RINGM
        plsc.store_scatter(buf, [pos], trash_pk)
    total = jnp.max(cur_v)
    padded = ((total + PB - 1) // PB) * PB

    def ffin(flushed):
        return lax.cond(flushed < padded, flush_block, lambda f: f, flushed)

    flushed = ffin(ffin(ffin(ffin(flushed))))
    cstage[pl.ds(0, LANES)] = jnp.zeros((LANES,), jnp.int32) + padded
    pltpu.sync_copy(cstage, counts_out.at[
        pl.ds(pl.multiple_of(wid * LANES, LANES), LANES)])


def _hop_body(pkl, counts, table, out_flat,
              pk0, pk1, idx0, idx1, lh0, lh1, rows0, rows1, cvm, accf,
              spk0, spk1, sg0, sg1):
    pk, idx, lh = [pk0, pk1], [idx0, idx1], [lh0, lh1]
    rows, spk, sg = [rows0, rows1], [spk0, spk1], [sg0, sg1]
    wid = _wid()
    lo = wid * RPW
    is_max = wid < NMAXW
    sgn = jnp.where(is_max, jnp.float32(1.0), jnp.float32(-1.0))
    s_splat = jnp.zeros((LANES,), jnp.float32) + sgn
    pos_splat = s_splat > 0.0
    lanes = lax.iota(jnp.int32, LANES)
    iotas = [lanes + c * LANES for c in range(NCH)]

    neg = jnp.full((LANES,), -jnp.inf, jnp.float32)

    def initb(r, _):
        for c in range(NCH):
            accf[pl.ds(r * DIM + c * LANES, LANES)] = neg
        return 0

    lax.fori_loop(0, RPW + 1, initb, 0)

    pltpu.sync_copy(counts.at[
        pl.ds(pl.multiple_of(wid * LANES, LANES), LANES)], cvm)
    cnt = jnp.max(cvm[pl.ds(0, LANES)])
    nb = cnt // B

    @pl.when(nb > 0)
    def _gather_accumulate():
        def issue_pk(b, s):
            bc = jnp.minimum(b, nb - 1)
            off = pl.multiple_of(wid * CAP + bc * B, B)
            pltpu.async_copy(pkl.at[pl.ds(off, B)], pk[s], spk[s])

        def wait_pk(s):
            pltpu.make_async_copy(pkl.at[pl.ds(0, B)], pk[s], spk[s]).wait()

        def unpack_gather(s):
            for i in range(B // LANES):
                v = pk[s][pl.ds(i * LANES, LANES)]
                idx[s][pl.ds(i * LANES, LANES)] = v & ((1 << SHIFT) - 1)
                lh[s][pl.ds(i * LANES, LANES)] = v >> SHIFT
            pltpu.async_copy(table.at[idx[s]], rows[s], sg[s])

        def wait_g(s):
            pltpu.make_async_copy(table.at[idx[s]], rows[s], sg[s]).wait()

        def acc_batch(s):
            def grp(kb, _):
                lv = lh[s][pl.ds(kb * LANES, LANES)]
                for lane in range(LANES):
                    hsp = lv[jnp.full((LANES,), lane, jnp.int32)]
                    base = hsp * DIM
                    row = kb * LANES + lane
                    for c in range(NCH):
                        addr = base + iotas[c]
                        a = plsc.load_gather(accf, [addr])
                        r = rows[s][row, pl.ds(c * LANES, LANES)] * s_splat
                        plsc.store_scatter(accf, [addr], jnp.maximum(a, r))
                return 0

            lax.fori_loop(0, B // LANES, grp, 0)

        issue_pk(0, 0)
        issue_pk(1, 1)
        wait_pk(0)
        unpack_gather(0)

        def gbody(g, _):
            for s in (0, 1):
                b = g * 2 + s
                wait_pk(s ^ 1)
                unpack_gather(s ^ 1)
                issue_pk(b + 2, s)
                wait_g(s)
                acc_batch(s)
            return 0

        lax.fori_loop(0, nb // 2, gbody, 0)
        wait_pk(1)
        wait_g(0)

    big_neg = jnp.full((LANES,), -3.0e38, jnp.float32)

    def fin(r, _):
        for c in range(NCH):
            sl = pl.ds(r * DIM + c * LANES, LANES)
            v = accf[sl]
            posv = jnp.maximum(v, 0.0)
            minv = jnp.where(v < big_neg, 0.0,
                             jnp.maximum(-v, 0.0))
            accf[sl] = jnp.where(pos_splat, posv, minv)
        return 0

    lax.fori_loop(0, RPW, fin, 0)
    pltpu.sync_copy(accf.at[pl.ds(0, RPW * DIM)],
                    out_flat.at[pl.ds(pl.multiple_of(lo * DIM, 64),
                                      RPW * DIM)])


_MESH = plsc.VectorSubcoreMesh(core_axis_name="c", subcore_axis_name="s")

_partition = functools.partial(
    pl.kernel,
    mesh=_MESH,
    compiler_params=pltpu.CompilerParams(needs_layout_passes=False),
    out_type=(
        jax.ShapeDtypeStruct((NWORK * CAP,), jnp.int32),
        jax.ShapeDtypeStruct((NWORK * LANES,), jnp.int32),
    ),
    scratch_types=[
        pltpu.VMEM((RING + LANES,), jnp.int32),
        pltpu.VMEM((LANES,), jnp.int32),
        pltpu.VMEM((CE,), jnp.int32),
        pltpu.VMEM((CE,), jnp.int32),
        pltpu.VMEM((CE,), jnp.int32),
        pltpu.VMEM((CE,), jnp.int32),
        pltpu.SemaphoreType.DMA,
        pltpu.SemaphoreType.DMA,
    ],
)(_partition_body)

_hop = functools.partial(
    pl.kernel,
    mesh=_MESH,
    compiler_params=pltpu.CompilerParams(needs_layout_passes=False),
    out_type=jax.ShapeDtypeStruct((NP * DIM,), jnp.float32),
    scratch_types=[
        pltpu.VMEM((B,), jnp.int32),
        pltpu.VMEM((B,), jnp.int32),
        pltpu.VMEM((B,), jnp.int32),
        pltpu.VMEM((B,), jnp.int32),
        pltpu.VMEM((B,), jnp.int32),
        pltpu.VMEM((B,), jnp.int32),
        pltpu.VMEM((B, DIM), jnp.float32),
        pltpu.VMEM((B, DIM), jnp.float32),
        pltpu.VMEM((LANES,), jnp.int32),
        pltpu.VMEM(((RPW + 1) * DIM,), jnp.float32),
        pltpu.SemaphoreType.DMA,
        pltpu.SemaphoreType.DMA,
        pltpu.SemaphoreType.DMA,
        pltpu.SemaphoreType.DMA,
    ],
)(_hop_body)


def kernel(visit_emb, visit_offset, ccs_emb, ccs_offset, icd_emb, icd_offset,
           cn_w1, cn_b1, cn_w2, cn_b2, t_w1, t_b1, t_w2, t_b2, graph):
    head = graph[0]
    tail = graph[1]

    pk_l, counts = _partition(head, tail)

    off0 = jnp.concatenate([visit_offset, ccs_offset, icd_offset], axis=0)
    off0 = jnp.concatenate(
        [off0, jnp.zeros((NP - NN, DIM), jnp.float32)], axis=0)

    off1 = _hop(pk_l, counts, off0)
    off2 = _hop(pk_l, counts, off1.reshape(NP, DIM))

    out_off = off2.reshape(NP, DIM)[:NV]
    out_emb = jnp.zeros((NV, DIM), jnp.float32)
    return out_emb, out_off


# speedup vs baseline: 15.1575x; 1.1264x over previous
import functools

import jax
import jax.numpy as jnp
from jax import lax
from jax.experimental import pallas as pl
from jax.experimental.pallas import tpu as pltpu
from jax.experimental.pallas import tpu_sc as plsc

NV = 6000
NC = 300
NI = 3700
NN = NV + NC + NI
E = 320000
DIM = 128
LANES = 16
NCH = DIM // LANES

NCORE = 2
NSUB = 16
NWORK = NCORE * NSUB
RPW = 315
NMAXW = (NV + NC) // RPW
NP = NWORK * RPW

CE = 2000
NVREG = CE // LANES
NCHUNK = E // CE
FLUSH = 2048
RING = 8192
RINGM = RING - 1
TR = RING
CAP = 158 * FLUSH
B = 32
PB = 2 * B
TRASH = RPW
SHIFT = 14


def _wid():
    return lax.axis_index("s") * NCORE + lax.axis_index("c")


def _partition_body(head_hbm, tail_hbm, pk_out, counts_out,
                    buf, cstage, h0, h1, t0, t1, sem0, sem1):
    hb, tb, sems = [h0, h1], [t0, t1], [sem0, sem1]
    wid = _wid()
    lo = wid * RPW
    hbm_base = wid * CAP
    lanes = lax.iota(jnp.int32, LANES)

    def issue(i, s):
        off = pl.multiple_of(jnp.minimum(i, NCHUNK - 1) * CE, CE)
        pltpu.async_copy(head_hbm.at[pl.ds(off, CE)], hb[s], sems[s])
        pltpu.async_copy(tail_hbm.at[pl.ds(off, CE)], tb[s], sems[s])

    def wait(s):
        pltpu.make_async_copy(head_hbm.at[pl.ds(0, CE)], hb[s], sems[s]).wait()
        pltpu.make_async_copy(tail_hbm.at[pl.ds(0, CE)], tb[s], sems[s]).wait()

    def flush_block(flushed):
        roff = pl.multiple_of(flushed & RINGM, FLUSH)
        pltpu.sync_copy(
            buf.at[pl.ds(roff, FLUSH)],
            pk_out.at[pl.ds(pl.multiple_of(hbm_base + flushed, FLUSH), FLUSH)])
        return flushed + FLUSH

    def chunk(i, s, carry):
        cur_v, flushed = carry
        wait(s)

        def vreg(j, cur_v):
            h = hb[s][pl.ds(j * LANES, LANES)]
            t = tb[s][pl.ds(j * LANES, LANES)]
            m = (h >= lo) & (h < lo + RPW)
            cs = plsc.cumsum(m.astype(jnp.int32))
            pos = jnp.where(m, (cur_v + cs - 1) & RINGM, TR)
            packed = t | ((h - lo) << SHIFT)
            plsc.store_scatter(buf, [pos], packed)
            return cur_v + plsc.all_reduce_population_count(m)

        cur_v = lax.fori_loop(0, NVREG, vreg, cur_v)
        issue(i + 2, s)
        cur_sc = jnp.max(cur_v)
        flushed = lax.cond(cur_sc - flushed >= FLUSH, flush_block,
                           lambda f: f, flushed)
        flushed = lax.cond(cur_sc - flushed >= FLUSH, flush_block,
                           lambda f: f, flushed)
        return (cur_v, flushed)

    issue(0, 0)
    issue(1, 1)

    def gbody(g, carry):
        for s in (0, 1):
            carry = chunk(g * 2 + s, s, carry)
        return carry

    cur_v, flushed = lax.fori_loop(
        0, NCHUNK // 2, gbody,
        (jnp.zeros((LANES,), jnp.int32), jnp.int32(0)))
    wait(0)
    wait(1)

    trash_pk = jnp.full((LANES,), TRASH << SHIFT, jnp.int32)
    for k in range(PB // LANES):
        pos = (cur_v + (k * LANES) + lanes) & RINGM
        plsc.store_scatter(buf, [pos], trash_pk)
    total = jnp.max(cur_v)
    padded = ((total + PB - 1) // PB) * PB

    def ffin(flushed):
        return lax.cond(flushed < padded, flush_block, lambda f: f, flushed)

    flushed = ffin(ffin(ffin(ffin(flushed))))
    cstage[pl.ds(0, LANES)] = jnp.zeros((LANES,), jnp.int32) + padded
    pltpu.sync_copy(cstage, counts_out.at[
        pl.ds(pl.multiple_of(wid * LANES, LANES), LANES)])


def _hop_body(pkl, counts, table, out_flat,
              pk0, pk1, idx0, idx1, lh0, lh1, rows0, rows1, cvm, accf, shtab,
              spk0, spk1, sg0, sg1, sst):
    pk, idx, lh = [pk0, pk1], [idx0, idx1], [lh0, lh1]
    rows, spk, sg = [rows0, rows1], [spk0, spk1], [sg0, sg1]
    wid = _wid()
    lo = wid * RPW
    is_max = wid < NMAXW
    pos_splat = jnp.zeros((LANES,), jnp.int32) + jnp.where(is_max, 1, 0) > 0
    lanes = lax.iota(jnp.int32, LANES)
    iotas = [lanes + c * LANES for c in range(NCH)]

    @pl.when(lax.axis_index("s") == 0)
    def _stage():
        pltpu.async_copy(table.at[pl.ds(0, NN)], shtab, sst).wait()

    ini = jnp.where(pos_splat, jnp.full((LANES,), -jnp.inf, jnp.float32),
                    jnp.full((LANES,), jnp.inf, jnp.float32))

    def initb(r, _):
        for c in range(NCH):
            accf[pl.ds(r * DIM + c * LANES, LANES)] = ini
        return 0

    lax.fori_loop(0, RPW + 1, initb, 0)

    pltpu.sync_copy(counts.at[
        pl.ds(pl.multiple_of(wid * LANES, LANES), LANES)], cvm)
    cnt = jnp.max(cvm[pl.ds(0, LANES)])
    nb = cnt // B

    plsc.subcore_barrier()

    @pl.when(nb > 0)
    def _gather_accumulate():
        def issue_pk(b, s):
            bc = jnp.minimum(b, nb - 1)
            off = pl.multiple_of(wid * CAP + bc * B, B)
            pltpu.async_copy(pkl.at[pl.ds(off, B)], pk[s], spk[s])

        def wait_pk(s):
            pltpu.make_async_copy(pkl.at[pl.ds(0, B)], pk[s], spk[s]).wait()

        def unpack_gather(s):
            for i in range(B // LANES):
                v = pk[s][pl.ds(i * LANES, LANES)]
                idx[s][pl.ds(i * LANES, LANES)] = v & ((1 << SHIFT) - 1)
                lh[s][pl.ds(i * LANES, LANES)] = v >> SHIFT
            pltpu.async_copy(shtab.at[idx[s]], rows[s], sg[s])

        def wait_g(s):
            pltpu.make_async_copy(shtab.at[idx[s]], rows[s], sg[s]).wait()

        def acc_batch(s):
            def run(op):
                def grp(kb, _):
                    lv = lh[s][pl.ds(kb * LANES, LANES)]
                    for lane in range(LANES):
                        hsp = lv[jnp.full((LANES,), lane, jnp.int32)]
                        base = hsp * DIM
                        row = kb * LANES + lane
                        for c in range(NCH):
                            addr = base + iotas[c]
                            a = plsc.load_gather(accf, [addr])
                            r = rows[s][row, pl.ds(c * LANES, LANES)]
                            plsc.store_scatter(accf, [addr], op(a, r))
                    return 0

                lax.fori_loop(0, B // LANES, grp, 0)
                return ()

            lax.cond(is_max, lambda: run(jnp.maximum), lambda: run(jnp.minimum))

        issue_pk(0, 0)
        issue_pk(1, 1)
        wait_pk(0)
        unpack_gather(0)

        def gbody(g, _):
            for s in (0, 1):
                b = g * 2 + s
                wait_pk(s ^ 1)
                unpack_gather(s ^ 1)
                issue_pk(b + 2, s)
                wait_g(s)
                acc_batch(s)
            return 0

        lax.fori_loop(0, nb // 2, gbody, 0)
        wait_pk(1)
        wait_g(0)

    big = jnp.full((LANES,), 3.0e38, jnp.float32)

    def fin(r, _):
        for c in range(NCH):
            sl = pl.ds(r * DIM + c * LANES, LANES)
            v = jnp.maximum(accf[sl], 0.0)
            accf[sl] = jnp.where(v >= big, 0.0, v)
        return 0

    lax.fori_loop(0, RPW, fin, 0)
    pltpu.sync_copy(accf.at[pl.ds(0, RPW * DIM)],
                    out_flat.at[pl.ds(pl.multiple_of(lo * DIM, 64),
                                      RPW * DIM)])


_MESH = plsc.VectorSubcoreMesh(core_axis_name="c", subcore_axis_name="s")

_partition = functools.partial(
    pl.kernel,
    mesh=_MESH,
    compiler_params=pltpu.CompilerParams(needs_layout_passes=False),
    out_type=(
        jax.ShapeDtypeStruct((NWORK * CAP,), jnp.int32),
        jax.ShapeDtypeStruct((NWORK * LANES,), jnp.int32),
    ),
    scratch_types=[
        pltpu.VMEM((RING + LANES,), jnp.int32),
        pltpu.VMEM((LANES,), jnp.int32),
        pltpu.VMEM((CE,), jnp.int32),
        pltpu.VMEM((CE,), jnp.int32),
        pltpu.VMEM((CE,), jnp.int32),
        pltpu.VMEM((CE,), jnp.int32),
        pltpu.SemaphoreType.DMA,
        pltpu.SemaphoreType.DMA,
    ],
)(_partition_body)

_hop = functools.partial(
    pl.kernel,
    mesh=_MESH,
    compiler_params=pltpu.CompilerParams(needs_layout_passes=False),
    out_type=jax.ShapeDtypeStruct((NP * DIM,), jnp.float32),
    scratch_types=[
        pltpu.VMEM((B,), jnp.int32),
        pltpu.VMEM((B,), jnp.int32),
        pltpu.VMEM((B,), jnp.int32),
        pltpu.VMEM((B,), jnp.int32),
        pltpu.VMEM((B,), jnp.int32),
        pltpu.VMEM((B,), jnp.int32),
        pltpu.VMEM((B, DIM), jnp.float32),
        pltpu.VMEM((B, DIM), jnp.float32),
        pltpu.VMEM((LANES,), jnp.int32),
        pltpu.VMEM(((RPW + 1) * DIM,), jnp.float32),
        pltpu.VMEM_SHARED((NN, DIM), jnp.float32),
        pltpu.SemaphoreType.DMA,
        pltpu.SemaphoreType.DMA,
        pltpu.SemaphoreType.DMA,
        pltpu.SemaphoreType.DMA,
        pltpu.SemaphoreType.DMA,
    ],
)(_hop_body)


def kernel(visit_emb, visit_offset, ccs_emb, ccs_offset, icd_emb, icd_offset,
           cn_w1, cn_b1, cn_w2, cn_b2, t_w1, t_b1, t_w2, t_b2, graph):
    head = graph[0]
    tail = graph[1]

    pk_l, counts = _partition(head, tail)

    off0 = jnp.concatenate([visit_offset, ccs_offset, icd_offset], axis=0)
    off0 = jnp.concatenate(
        [off0, jnp.zeros((NP - NN, DIM), jnp.float32)], axis=0)

    off1 = _hop(pk_l, counts, off0)
    off2 = _hop(pk_l, counts, off1.reshape(NP, DIM))

    out_off = off2.reshape(NP, DIM)[:NV]
    out_emb = jnp.zeros((NV, DIM), jnp.float32)
    return out_emb, out_off
